# Initial kernel scaffold; baseline (speedup 1.0000x reference)
import jax
import jax.numpy as jnp
from jax.experimental import pallas as pl


def kernel(Z, edge_index, edge_weight, alpha):
    raise NotImplementedError("write your pallas kernel here")


# speedup vs baseline: 2.0759x; 2.0759x over previous
import functools

import jax
import jax.numpy as jnp
from jax import lax
from jax.experimental import pallas as pl
from jax.experimental.pallas import tpu as pltpu
from jax.experimental.pallas import tpu_sc as plsc

N = 10000
D = 128
NC = 2
NS = 16
NW = NC * NS
C = 128
CPW = 80
EPW = C * CPW
EPAD = EPW * NW
RPS = 624
TAIL0 = RPS * NS
TAILN = N - TAIL0


def _sc_spmm(Z, src2, dst2, wb):
    mesh = plsc.VectorSubcoreMesh(core_axis_name="c", subcore_axis_name="s",
                                  num_cores=NC, num_subcores=NS)

    @functools.partial(
        pl.kernel,
        out_type=jax.ShapeDtypeStruct((NC, N, D), jnp.float32),
        mesh=mesh,
        compiler_params=pltpu.CompilerParams(use_tc_tiling_on_sc=False),
        scratch_types=dict(
            src_v=pltpu.VMEM((CPW, C), jnp.int32),
            dst_v=pltpu.VMEM((CPW, C), jnp.int32),
            wb_v=pltpu.VMEM((C, 16), jnp.float32),
            rows_v=pltpu.VMEM((C, D), jnp.float32),
            acc=pltpu.VMEM_SHARED((N, D), jnp.float32),
        ),
    )
    def k(z_hbm, src_hbm, dst_hbm, wb_hbm, out_hbm,
          src_v, dst_v, wb_v, rows_v, acc):
        cid = lax.axis_index("c")
        sid = lax.axis_index("s")
        wid = sid * NC + cid

        def zrow(e, carry):
            for j in range(D // 16):
                rows_v[e, pl.ds(16 * j, 16)] = jnp.zeros((16,), jnp.float32)
            return carry
        lax.fori_loop(0, C, zrow, 0)
        for off in range(0, RPS, C):
            sz = min(C, RPS - off)
            pltpu.sync_copy(rows_v.at[pl.ds(0, sz)],
                            acc.at[pl.ds(sid * RPS + off, sz)])
        @pl.when(sid == NS - 1)
        def _zero_tail():
            pltpu.sync_copy(rows_v.at[pl.ds(0, TAILN)],
                            acc.at[pl.ds(TAIL0, TAILN)])
        plsc.subcore_barrier()

        pltpu.sync_copy(src_hbm.at[pl.ds(wid * CPW, CPW)], src_v)
        pltpu.sync_copy(dst_hbm.at[pl.ds(wid * CPW, CPW)], dst_v)

        def chunk(g, carry):
            pltpu.sync_copy(wb_hbm.at[pl.ds(wid * EPW + g * C, C)], wb_v)
            pltpu.sync_copy(z_hbm.at[src_v.at[g]], rows_v)
            def scale(e, c2):
                wv = wb_v[e]
                for j in range(D // 16):
                    rows_v[e, pl.ds(16 * j, 16)] = (
                        rows_v[e, pl.ds(16 * j, 16)] * wv)
                return c2
            lax.fori_loop(0, C, scale, 0)
            pltpu.sync_copy(rows_v, acc.at[dst_v.at[g]], add=True)
            return carry
        lax.fori_loop(0, CPW, chunk, 0)
        plsc.subcore_barrier()

        for off in range(0, RPS, C):
            sz = min(C, RPS - off)
            pltpu.sync_copy(acc.at[pl.ds(sid * RPS + off, sz)],
                            rows_v.at[pl.ds(0, sz)])
            pltpu.sync_copy(rows_v.at[pl.ds(0, sz)],
                            out_hbm.at[cid, pl.ds(sid * RPS + off, sz)])
        @pl.when(sid == NS - 1)
        def _write_tail():
            pltpu.sync_copy(acc.at[pl.ds(TAIL0, TAILN)], rows_v.at[pl.ds(0, TAILN)])
            pltpu.sync_copy(rows_v.at[pl.ds(0, TAILN)],
                            out_hbm.at[cid, pl.ds(TAIL0, TAILN)])

    return k(Z, src2, dst2, wb)


def _merge(parts, alpha):
    def mk(a_ref, p_ref, o_ref):
        o_ref[...] = a_ref[0, 0] * (p_ref[0] + p_ref[1])

    return pl.pallas_call(
        mk,
        grid=(10,),
        in_specs=[
            pl.BlockSpec(memory_space=pltpu.SMEM),
            pl.BlockSpec((NC, N // 10, D), lambda i: (0, i, 0)),
        ],
        out_specs=pl.BlockSpec((N // 10, D), lambda i: (i, 0)),
        out_shape=jax.ShapeDtypeStruct((N, D), jnp.float32),
    )(alpha.reshape(1, 1), parts)


def kernel(Z, edge_index, edge_weight, alpha):
    src = edge_index[0].astype(jnp.int32)
    dst = edge_index[1].astype(jnp.int32)
    w = edge_weight.astype(jnp.float32)
    pad = EPAD - src.shape[0]
    src2 = jnp.pad(src, (0, pad)).reshape(EPAD // C, C)
    dst2 = jnp.pad(dst, (0, pad)).reshape(EPAD // C, C)
    wb = jnp.broadcast_to(jnp.pad(w, (0, pad))[:, None], (EPAD, 16))
    parts = _sc_spmm(Z, src2, dst2, wb)
    return _merge(parts, alpha)


# speedup vs baseline: 4.2654x; 2.0547x over previous
import functools

import jax
import jax.numpy as jnp
from jax import lax
from jax.experimental import pallas as pl
from jax.experimental.pallas import tpu as pltpu
from jax.experimental.pallas import tpu_sc as plsc

N = 10000
D = 128
HD = D // 2
NC = 2
NS = 16
C = 128
CPT = 159
EPT = C * CPT
EPAD = EPT * NS
NB = 3
RPS = 624
TAIL0 = RPS * NS
TAILN = N - TAIL0


def _sc_spmm(z3, src2, dst2, w1, a8):
    mesh = plsc.VectorSubcoreMesh(core_axis_name="c", subcore_axis_name="s",
                                  num_cores=NC, num_subcores=NS)

    @functools.partial(
        pl.kernel,
        out_type=jax.ShapeDtypeStruct((NC, N, HD), jnp.float32),
        mesh=mesh,
        compiler_params=pltpu.CompilerParams(use_tc_tiling_on_sc=False),
        scratch_types=dict(
            src_v=pltpu.VMEM((CPT, C), jnp.int32),
            dst_v=pltpu.VMEM((CPT, C), jnp.int32),
            w_v=pltpu.VMEM((CPT, C), jnp.float32),
            a_v=pltpu.VMEM((16,), jnp.float32),
            rows=[pltpu.VMEM((C, HD), jnp.float32) for _ in range(NB)],
            gsem=[pltpu.SemaphoreType.DMA for _ in range(NB)],
            ssem=[pltpu.SemaphoreType.DMA for _ in range(NB)],
            acc=pltpu.VMEM_SHARED((N, HD), jnp.float32),
        ),
    )
    def k(z_hbm, src_hbm, dst_hbm, w_hbm, a_hbm, out_hbm,
          src_v, dst_v, w_v, a_v, rows, gsem, ssem, acc):
        cid = lax.axis_index("c")
        sid = lax.axis_index("s")

        def zrow(e, carry):
            for j in range(HD // 16):
                rows[0][e, pl.ds(16 * j, 16)] = jnp.zeros((16,), jnp.float32)
            return carry
        lax.fori_loop(0, C, zrow, 0)
        for off in range(0, RPS, C):
            sz = min(C, RPS - off)
            pltpu.sync_copy(rows[0].at[pl.ds(0, sz)],
                            acc.at[pl.ds(sid * RPS + off, sz)])
        @pl.when(sid == NS - 1)
        def _zero_tail():
            pltpu.sync_copy(rows[0].at[pl.ds(0, TAILN)],
                            acc.at[pl.ds(TAIL0, TAILN)])
        plsc.subcore_barrier()

        pltpu.sync_copy(src_hbm.at[pl.ds(sid * CPT, CPT)], src_v)
        pltpu.sync_copy(dst_hbm.at[pl.ds(sid * CPT, CPT)], dst_v)
        pltpu.sync_copy(w_hbm.at[pl.ds(sid * CPT, CPT)], w_v)
        pltpu.sync_copy(a_hbm, a_v)
        av16 = a_v[...]
        def wscale(i, carry):
            for j in range(C // 16):
                w_v[i, pl.ds(16 * j, 16)] = w_v[i, pl.ds(16 * j, 16)] * av16
            return carry
        lax.fori_loop(0, CPT, wscale, 0)

        def start_gather(b, g):
            pltpu.async_copy(z_hbm.at[cid].at[src_v.at[g]], rows[b], gsem[b])

        def wait_gather(b, g):
            pltpu.make_async_copy(z_hbm.at[cid].at[src_v.at[g]], rows[b],
                                  gsem[b]).wait()

        def start_scatter(b, g):
            pltpu.async_copy(rows[b], acc.at[dst_v.at[g]], ssem[b],
                             add=True)

        def wait_scatter(b, g):
            pltpu.make_async_copy(rows[b], acc.at[dst_v.at[g]],
                                  ssem[b]).wait()

        start_gather(0, 0)
        start_gather(1, 1)

        def outer(t, carry):
            for b in range(NB):
                g = t * NB + b
                wait_gather(b, g)
                def scale(eg, c2):
                    wv16 = w_v[g, pl.ds(16 * eg, 16)]
                    for l in range(16):
                        e = 16 * eg + l
                        for j in range(HD // 16):
                            rows[b][e, pl.ds(16 * j, 16)] = (
                                rows[b][e, pl.ds(16 * j, 16)] * wv16[l])
                    return c2
                lax.fori_loop(0, C // 16, scale, 0)
                start_scatter(b, g)
                bp = (b + 2) % NB
                if b == 0:
                    @pl.when(t > 0)
                    def _drain0():
                        wait_scatter(bp, g - 1)
                else:
                    wait_scatter(bp, g - 1)
                @pl.when(g + 2 < CPT)
                def _next():
                    start_gather(bp, g + 2)
            return carry
        lax.fori_loop(0, CPT // NB, outer, 0)
        wait_scatter((CPT - 1) % NB, CPT - 1)
        plsc.subcore_barrier()

        for off in range(0, RPS, C):
            sz = min(C, RPS - off)
            pltpu.sync_copy(acc.at[pl.ds(sid * RPS + off, sz)],
                            rows[0].at[pl.ds(0, sz)])
            pltpu.sync_copy(rows[0].at[pl.ds(0, sz)],
                            out_hbm.at[cid].at[pl.ds(sid * RPS + off, sz)])
        @pl.when(sid == NS - 1)
        def _write_tail():
            pltpu.sync_copy(acc.at[pl.ds(TAIL0, TAILN)],
                            rows[1].at[pl.ds(0, TAILN)])
            pltpu.sync_copy(rows[1].at[pl.ds(0, TAILN)],
                            out_hbm.at[cid].at[pl.ds(TAIL0, TAILN)])

    return k(z3, src2, dst2, w1, a8)


def kernel(Z, edge_index, edge_weight, alpha):
    src = edge_index[0].astype(jnp.int32)
    dst = edge_index[1].astype(jnp.int32)
    w = edge_weight.astype(jnp.float32)
    pad = EPAD - src.shape[0]
    src2 = jnp.pad(src, (0, pad)).reshape(EPAD // C, C)
    dst2 = jnp.pad(dst, (0, pad)).reshape(EPAD // C, C)
    w1 = jnp.pad(w, (0, pad)).reshape(EPAD // C, C)
    a16 = jnp.broadcast_to(alpha.astype(jnp.float32)[None], (16,))
    z_t = jnp.swapaxes(Z.reshape(N, NC, HD), 0, 1)
    out_t = _sc_spmm(z_t, src2, dst2, w1, a16)
    return jnp.swapaxes(out_t, 0, 1).reshape(N, D)
